# baseline (speedup 1.0000x reference)
import math

import jax
import jax.numpy as jnp
from jax.experimental import pallas as pl
from jax.experimental.pallas import tpu as pltpu

_N, _D, _E, _H, _C, _TOPK = 2048, 768, 16, 128, 10, 4
_EH = _E * _H
_EC = _E * _C
_EPS = 1e-5
_BN = 512


def _moe_block_kernel(x_ref, rw_ref, rb_ref, lnst_ref, lnb_ref, w1_ref, b1_ref,
                      w2_ref, b2_ref, weighted_ref, all_probs_ref, gate_ref,
                      w1s_ref, b1e_ref, w2bd_ref, b2c_ref, bt_ref, b_ref, g_ref):
    @pl.when(pl.program_id(0) == 0)
    def _fold():
        w2bd_ref[...] = jnp.zeros((_EH, _EC), jnp.bfloat16)
        for e in range(_E):
            s = lnst_ref[:, e:e + 1]
            w1s_ref[:, e * _H:(e + 1) * _H] = (s * w1_ref[e]).astype(jnp.bfloat16)
            tb = jnp.dot(lnb_ref[e].reshape(1, _D), w1_ref[e],
                         preferred_element_type=jnp.float32)
            b1e_ref[:, e * _H:(e + 1) * _H] = tb + b1_ref[e][None, :]
            w2bd_ref[e * _H:(e + 1) * _H, e * _C:(e + 1) * _C] = (
                w2_ref[e].astype(jnp.bfloat16))
            b2c_ref[:, e * _C:(e + 1) * _C] = b2_ref[e][None, :]
        seg_of_lane = jax.lax.broadcasted_iota(jnp.int32, (_EC, _E), 0) // _C
        ecol = jax.lax.broadcasted_iota(jnp.int32, (_EC, _E), 1)
        bt_ref[...] = (seg_of_lane == ecol).astype(jnp.float32)
        seg_r = jax.lax.broadcasted_iota(jnp.int32, (_E, _EC), 0)
        lane_r = jax.lax.broadcasted_iota(jnp.int32, (_E, _EC), 1) // _C
        b_ref[...] = (seg_r == lane_r).astype(jnp.float32)
        lane_c = jax.lax.broadcasted_iota(jnp.int32, (_EC, _C), 0) % _C
        ccol = jax.lax.broadcasted_iota(jnp.int32, (_EC, _C), 1)
        g_ref[...] = (lane_c == ccol).astype(jnp.float32)

    x = x_ref[...]

    gl = jnp.dot(x, rw_ref[...], preferred_element_type=jnp.float32)
    gl = gl + rb_ref[...]
    gl = gl - jnp.max(gl, axis=-1, keepdims=True)
    ge = jnp.exp(gl)
    gp = ge / jnp.sum(ge, axis=-1, keepdims=True)
    gate_ref[...] = gp

    gpt = gp.T
    erow = jax.lax.broadcasted_iota(jnp.int32, (_E, _BN), 0)
    rankt = jnp.zeros((_E, _BN), dtype=jnp.int32)
    for ep in range(_E):
        row = gpt[ep:ep + 1, :]
        beats = (row > gpt) | ((row == gpt) & (ep < erow))
        rankt = rankt + beats.astype(jnp.int32)
    wsel = jnp.where(rankt < _TOPK, gpt, 0.0).T
    wsel = wsel / jnp.sum(wsel, axis=-1, keepdims=True)

    mu = jnp.mean(x, axis=-1, keepdims=True)
    xc = x - mu
    var = jnp.mean(xc * xc, axis=-1, keepdims=True)
    xn = xc * jax.lax.rsqrt(var + _EPS)

    h = jnp.dot(xn.astype(jnp.bfloat16), w1s_ref[...],
                preferred_element_type=jnp.float32)
    h = h + b1e_ref[...]
    h = 0.5 * h * (1.0 + jax.lax.erf(h * (1.0 / math.sqrt(2.0))))
    lo = jnp.dot(h.astype(jnp.bfloat16), w2bd_ref[...],
                 preferred_element_type=jnp.float32)
    lo = lo + b2c_ref[...]

    m = jnp.max(lo, axis=-1, keepdims=True)
    p = jnp.exp(lo - m)
    ssum = jnp.dot(p, bt_ref[...], preferred_element_type=jnp.float32)
    sr = 1.0 / ssum
    probs = p * jnp.dot(sr, b_ref[...], preferred_element_type=jnp.float32)
    for e in range(_E):
        all_probs_ref[e] = probs[:, e * _C:(e + 1) * _C]

    cw160 = jnp.dot(wsel * sr, b_ref[...],
                    preferred_element_type=jnp.float32)
    weighted_ref[...] = jnp.dot(cw160 * p, g_ref[...],
                                preferred_element_type=jnp.float32)


@jax.jit
def kernel(x, router_w, router_b, ln_scale, ln_bias, w1, b1, w2, b2):
    rb2 = router_b.reshape(1, _E)
    grid = (_N // _BN,)
    out_shapes = (
        jax.ShapeDtypeStruct((_N, _C), jnp.float32),
        jax.ShapeDtypeStruct((_E, _N, _C), jnp.float32),
        jax.ShapeDtypeStruct((_N, _E), jnp.float32),
    )
    in_specs = [
        pl.BlockSpec((_BN, _D), lambda i: (i, 0)),
        pl.BlockSpec((_D, _E), lambda i: (0, 0)),
        pl.BlockSpec((1, _E), lambda i: (0, 0)),
        pl.BlockSpec((_D, _E), lambda i: (0, 0)),
        pl.BlockSpec((_E, _D), lambda i: (0, 0)),
        pl.BlockSpec((_E, _D, _H), lambda i: (0, 0, 0)),
        pl.BlockSpec((_E, _H), lambda i: (0, 0)),
        pl.BlockSpec((_E, _H, _C), lambda i: (0, 0, 0)),
        pl.BlockSpec((_E, _C), lambda i: (0, 0)),
    ]
    out_specs = (
        pl.BlockSpec((_BN, _C), lambda i: (i, 0)),
        pl.BlockSpec((_E, _BN, _C), lambda i: (0, i, 0)),
        pl.BlockSpec((_BN, _E), lambda i: (i, 0)),
    )
    scratch_shapes = [
        pltpu.VMEM((_D, _EH), jnp.bfloat16),
        pltpu.VMEM((1, _EH), jnp.float32),
        pltpu.VMEM((_EH, _EC), jnp.bfloat16),
        pltpu.VMEM((1, _EC), jnp.float32),
        pltpu.VMEM((_EC, _E), jnp.float32),
        pltpu.VMEM((_E, _EC), jnp.float32),
        pltpu.VMEM((_EC, _C), jnp.float32),
    ]
    weighted, all_probs, gate_probs = pl.pallas_call(
        _moe_block_kernel,
        grid=grid,
        in_specs=in_specs,
        out_specs=out_specs,
        out_shape=out_shapes,
        scratch_shapes=scratch_shapes,
    )(x, router_w, rb2, ln_scale.T, ln_bias, w1, b1, w2, b2)
    return weighted, all_probs, gate_probs


# speedup vs baseline: 1.0382x; 1.0382x over previous
import math

import jax
import jax.numpy as jnp
from jax.experimental import pallas as pl
from jax.experimental.pallas import tpu as pltpu

_N, _D, _E, _H, _C, _TOPK = 2048, 768, 16, 128, 10, 4
_EH = _E * _H
_EC = _E * _C
_EPS = 1e-5
_BN = 512


def _moe_block_kernel(x_ref, rw_ref, rb_ref, lns_ref, lnb_ref, w1_ref, b1_ref,
                      w2_ref, b2_ref, weighted_ref, all_probs_ref, gate_ref,
                      w1s_ref, b1e_ref, w2bd_ref, b2c_ref, bt_ref, b_ref, g_ref):
    @pl.when(pl.program_id(0) == 0)
    def _fold():
        w2bd_ref[...] = jnp.zeros((_EH, _EC), jnp.bfloat16)
        lnst = lns_ref[...].T
        for e in range(_E):
            s = lnst[:, e:e + 1]
            w1s_ref[:, e * _H:(e + 1) * _H] = (s * w1_ref[e]).astype(jnp.bfloat16)
            tb = jnp.dot(lnb_ref[e].reshape(1, _D), w1_ref[e],
                         preferred_element_type=jnp.float32)
            b1e_ref[:, e * _H:(e + 1) * _H] = tb + b1_ref[e][None, :]
            w2bd_ref[e * _H:(e + 1) * _H, e * _C:(e + 1) * _C] = (
                w2_ref[e].astype(jnp.bfloat16))
            b2c_ref[:, e * _C:(e + 1) * _C] = b2_ref[e][None, :]
        seg_of_lane = jax.lax.broadcasted_iota(jnp.int32, (_EC, _E), 0) // _C
        ecol = jax.lax.broadcasted_iota(jnp.int32, (_EC, _E), 1)
        bt_ref[...] = (seg_of_lane == ecol).astype(jnp.float32)
        seg_r = jax.lax.broadcasted_iota(jnp.int32, (_E, _EC), 0)
        lane_r = jax.lax.broadcasted_iota(jnp.int32, (_E, _EC), 1) // _C
        b_ref[...] = (seg_r == lane_r).astype(jnp.float32)
        lane_c = jax.lax.broadcasted_iota(jnp.int32, (_EC, _C), 0) % _C
        ccol = jax.lax.broadcasted_iota(jnp.int32, (_EC, _C), 1)
        g_ref[...] = (lane_c == ccol).astype(jnp.float32)

    x = x_ref[...]

    gl = jnp.dot(x, rw_ref[...], preferred_element_type=jnp.float32)
    gl = gl + rb_ref[...]
    gl = gl - jnp.max(gl, axis=-1, keepdims=True)
    ge = jnp.exp(gl)
    gp = ge / jnp.sum(ge, axis=-1, keepdims=True)
    gate_ref[...] = gp

    gpt = gp.T
    erow = jax.lax.broadcasted_iota(jnp.int32, (_E, _BN), 0)
    rankt = jnp.zeros((_E, _BN), dtype=jnp.int32)
    for ep in range(_E):
        row = gpt[ep:ep + 1, :]
        beats = (row > gpt) | ((row == gpt) & (ep < erow))
        rankt = rankt + beats.astype(jnp.int32)
    wsel = jnp.where(rankt < _TOPK, gpt, 0.0).T
    wsel = wsel / jnp.sum(wsel, axis=-1, keepdims=True)

    mu = jnp.mean(x, axis=-1, keepdims=True)
    xc = x - mu
    var = jnp.mean(xc * xc, axis=-1, keepdims=True)
    xn = xc * jax.lax.rsqrt(var + _EPS)

    h = jnp.dot(xn.astype(jnp.bfloat16), w1s_ref[...],
                preferred_element_type=jnp.float32)
    h = h + b1e_ref[...]
    h = 0.5 * h * (1.0 + jax.lax.erf(h * (1.0 / math.sqrt(2.0))))
    lo = jnp.dot(h.astype(jnp.bfloat16), w2bd_ref[...],
                 preferred_element_type=jnp.float32)
    lo = lo + b2c_ref[...]

    m = jnp.max(lo, axis=-1, keepdims=True)
    p = jnp.exp(lo - m)
    ssum = jnp.dot(p, bt_ref[...], preferred_element_type=jnp.float32)
    sr = 1.0 / ssum
    probs = p * jnp.dot(sr, b_ref[...], preferred_element_type=jnp.float32)
    for e in range(_E):
        all_probs_ref[e] = probs[:, e * _C:(e + 1) * _C]

    cw160 = jnp.dot(wsel * sr, b_ref[...],
                    preferred_element_type=jnp.float32)
    weighted_ref[...] = jnp.dot(cw160 * p, g_ref[...],
                                preferred_element_type=jnp.float32)


@jax.jit
def kernel(x, router_w, router_b, ln_scale, ln_bias, w1, b1, w2, b2):
    rb2 = router_b.reshape(1, _E)
    grid = (_N // _BN,)
    out_shapes = (
        jax.ShapeDtypeStruct((_N, _C), jnp.float32),
        jax.ShapeDtypeStruct((_E, _N, _C), jnp.float32),
        jax.ShapeDtypeStruct((_N, _E), jnp.float32),
    )
    in_specs = [
        pl.BlockSpec((_BN, _D), lambda i: (i, 0)),
        pl.BlockSpec((_D, _E), lambda i: (0, 0)),
        pl.BlockSpec((1, _E), lambda i: (0, 0)),
        pl.BlockSpec((_E, _D), lambda i: (0, 0)),
        pl.BlockSpec((_E, _D), lambda i: (0, 0)),
        pl.BlockSpec((_E, _D, _H), lambda i: (0, 0, 0)),
        pl.BlockSpec((_E, _H), lambda i: (0, 0)),
        pl.BlockSpec((_E, _H, _C), lambda i: (0, 0, 0)),
        pl.BlockSpec((_E, _C), lambda i: (0, 0)),
    ]
    out_specs = (
        pl.BlockSpec((_BN, _C), lambda i: (i, 0)),
        pl.BlockSpec((_E, _BN, _C), lambda i: (0, i, 0)),
        pl.BlockSpec((_BN, _E), lambda i: (i, 0)),
    )
    scratch_shapes = [
        pltpu.VMEM((_D, _EH), jnp.bfloat16),
        pltpu.VMEM((1, _EH), jnp.float32),
        pltpu.VMEM((_EH, _EC), jnp.bfloat16),
        pltpu.VMEM((1, _EC), jnp.float32),
        pltpu.VMEM((_EC, _E), jnp.float32),
        pltpu.VMEM((_E, _EC), jnp.float32),
        pltpu.VMEM((_EC, _C), jnp.float32),
    ]
    weighted, all_probs, gate_probs = pl.pallas_call(
        _moe_block_kernel,
        grid=grid,
        in_specs=in_specs,
        out_specs=out_specs,
        out_shape=out_shapes,
        scratch_shapes=scratch_shapes,
    )(x, router_w, rb2, ln_scale, ln_bias, w1, b1, w2, b2)
    return weighted, all_probs, gate_probs


# speedup vs baseline: 1.0452x; 1.0068x over previous
import math

import jax
import jax.numpy as jnp
from jax.experimental import pallas as pl
from jax.experimental.pallas import tpu as pltpu

_N, _D, _E, _H, _C, _TOPK = 2048, 768, 16, 128, 10, 4
_EH = _E * _H
_EC = _E * _C
_EPS = 1e-5
_BN = 512


def _moe_block_kernel(x_ref, rw_ref, rb_ref, lns_ref, lnb_ref, w1_ref, b1_ref,
                      w2_ref, b2_ref, weighted_ref, all_probs_ref, gate_ref,
                      w1s_ref, b1e_ref, w2bd_ref, b2c_ref, bt_ref, b_ref, g_ref):
    @pl.when(pl.program_id(0) == 0)
    def _fold():
        w2bd_ref[...] = jnp.zeros((_EH, _EC), jnp.bfloat16)
        lnst = lns_ref[...].T
        for e in range(_E):
            s = lnst[:, e:e + 1]
            w1s_ref[:, e * _H:(e + 1) * _H] = (s * w1_ref[e]).astype(jnp.bfloat16)
            tb = jnp.dot(lnb_ref[e].reshape(1, _D), w1_ref[e],
                         preferred_element_type=jnp.float32)
            b1e_ref[:, e * _H:(e + 1) * _H] = tb + b1_ref[e][None, :]
            w2bd_ref[e * _H:(e + 1) * _H, e * _C:(e + 1) * _C] = (
                w2_ref[e].astype(jnp.bfloat16))
            b2c_ref[:, e * _C:(e + 1) * _C] = b2_ref[e][None, :]
        seg_of_lane = jax.lax.broadcasted_iota(jnp.int32, (_EC, _E), 0) // _C
        ecol = jax.lax.broadcasted_iota(jnp.int32, (_EC, _E), 1)
        bt_ref[...] = (seg_of_lane == ecol).astype(jnp.float32)
        seg_r = jax.lax.broadcasted_iota(jnp.int32, (_E, _EC), 0)
        lane_r = jax.lax.broadcasted_iota(jnp.int32, (_E, _EC), 1) // _C
        b_ref[...] = (seg_r == lane_r).astype(jnp.float32)
        lane_c = jax.lax.broadcasted_iota(jnp.int32, (_EC, _C), 0) % _C
        ccol = jax.lax.broadcasted_iota(jnp.int32, (_EC, _C), 1)
        g_ref[...] = (lane_c == ccol).astype(jnp.float32)

    _HB = _BN // 2
    for half in range(2):
        sl = pl.ds(half * _HB, _HB)
        x = x_ref[sl, :]

        gl = jnp.dot(x, rw_ref[...], preferred_element_type=jnp.float32)
        gl = gl + rb_ref[...]
        gl = gl - jnp.max(gl, axis=-1, keepdims=True)
        ge = jnp.exp(gl)
        gp = ge / jnp.sum(ge, axis=-1, keepdims=True)
        gate_ref[sl, :] = gp

        gpt = gp.T
        erow = jax.lax.broadcasted_iota(jnp.int32, (_E, _HB), 0)
        rankt = jnp.zeros((_E, _HB), dtype=jnp.int32)
        for ep in range(_E):
            row = gpt[ep:ep + 1, :]
            beats = (row > gpt) | ((row == gpt) & (ep < erow))
            rankt = rankt + beats.astype(jnp.int32)
        wsel = jnp.where(rankt < _TOPK, gpt, 0.0).T
        wsel = wsel / jnp.sum(wsel, axis=-1, keepdims=True)

        mu = jnp.mean(x, axis=-1, keepdims=True)
        xc = x - mu
        var = jnp.mean(xc * xc, axis=-1, keepdims=True)
        xn = xc * jax.lax.rsqrt(var + _EPS)

        h = jnp.dot(xn.astype(jnp.bfloat16), w1s_ref[...],
                    preferred_element_type=jnp.float32)
        h = h + b1e_ref[...]
        h = 0.5 * h * (1.0 + jax.lax.erf(h * (1.0 / math.sqrt(2.0))))
        lo = jnp.dot(h.astype(jnp.bfloat16), w2bd_ref[...],
                     preferred_element_type=jnp.float32)
        lo = lo + b2c_ref[...]

        m = jnp.max(lo, axis=-1, keepdims=True)
        p = jnp.exp(lo - m)
        ssum = jnp.dot(p, bt_ref[...], preferred_element_type=jnp.float32)
        sr = 1.0 / ssum
        probs = p * jnp.dot(sr, b_ref[...], preferred_element_type=jnp.float32)
        for e in range(_E):
            all_probs_ref[e, sl, :] = probs[:, e * _C:(e + 1) * _C]

        cw160 = jnp.dot(wsel * sr, b_ref[...],
                        preferred_element_type=jnp.float32)
        weighted_ref[sl, :] = jnp.dot(cw160 * p, g_ref[...],
                                      preferred_element_type=jnp.float32)


@jax.jit
def kernel(x, router_w, router_b, ln_scale, ln_bias, w1, b1, w2, b2):
    rb2 = router_b.reshape(1, _E)
    grid = (_N // _BN,)
    out_shapes = (
        jax.ShapeDtypeStruct((_N, _C), jnp.float32),
        jax.ShapeDtypeStruct((_E, _N, _C), jnp.float32),
        jax.ShapeDtypeStruct((_N, _E), jnp.float32),
    )
    in_specs = [
        pl.BlockSpec((_BN, _D), lambda i: (i, 0)),
        pl.BlockSpec((_D, _E), lambda i: (0, 0)),
        pl.BlockSpec((1, _E), lambda i: (0, 0)),
        pl.BlockSpec((_E, _D), lambda i: (0, 0)),
        pl.BlockSpec((_E, _D), lambda i: (0, 0)),
        pl.BlockSpec((_E, _D, _H), lambda i: (0, 0, 0)),
        pl.BlockSpec((_E, _H), lambda i: (0, 0)),
        pl.BlockSpec((_E, _H, _C), lambda i: (0, 0, 0)),
        pl.BlockSpec((_E, _C), lambda i: (0, 0)),
    ]
    out_specs = (
        pl.BlockSpec((_BN, _C), lambda i: (i, 0)),
        pl.BlockSpec((_E, _BN, _C), lambda i: (0, i, 0)),
        pl.BlockSpec((_BN, _E), lambda i: (i, 0)),
    )
    scratch_shapes = [
        pltpu.VMEM((_D, _EH), jnp.bfloat16),
        pltpu.VMEM((1, _EH), jnp.float32),
        pltpu.VMEM((_EH, _EC), jnp.bfloat16),
        pltpu.VMEM((1, _EC), jnp.float32),
        pltpu.VMEM((_EC, _E), jnp.float32),
        pltpu.VMEM((_E, _EC), jnp.float32),
        pltpu.VMEM((_EC, _C), jnp.float32),
    ]
    weighted, all_probs, gate_probs = pl.pallas_call(
        _moe_block_kernel,
        grid=grid,
        in_specs=in_specs,
        out_specs=out_specs,
        out_shape=out_shapes,
        scratch_shapes=scratch_shapes,
    )(x, router_w, rb2, ln_scale, ln_bias, w1, b1, w2, b2)
    return weighted, all_probs, gate_probs


# speedup vs baseline: 1.9586x; 1.8739x over previous
import math

import jax
import jax.numpy as jnp
from jax.experimental import pallas as pl
from jax.experimental.pallas import tpu as pltpu

_N, _D, _E, _H, _C, _TOPK = 2048, 768, 16, 128, 10, 4
_EH = _E * _H
_EC = _E * _C
_EPS = 1e-5
_BN = 512
_HB = _BN // 2


def _moe_block_kernel(x_ref, rwt_ref, rb_ref, lns_ref, lnb_ref, w1_ref, b1_ref,
                      w2t_ref, b2_ref, wt_ref, apt_ref, gpt_out_ref,
                      w1s_ref, b1e_ref, w2bd_ref, b2c_ref, bt_ref, b_ref, g_ref):
    @pl.when(pl.program_id(0) == 0)
    def _fold():
        lnst = lns_ref[...].T
        lane = jax.lax.broadcasted_iota(jnp.int32, (_C, _EC), 1)
        crow = jax.lax.broadcasted_iota(jnp.int32, (_C, _EC), 0)
        for e in range(_E):
            s = lnst[:, e:e + 1]
            w1s_ref[:, e * _H:(e + 1) * _H] = (s * w1_ref[e]).astype(jnp.bfloat16)
            tb = jnp.dot(lnb_ref[e].reshape(1, _D), w1_ref[e],
                         preferred_element_type=jnp.float32)
            b1e_ref[:, e * _H:(e + 1) * _H] = tb + b1_ref[e][None, :]
            scat = (lane == crow * _E + e).astype(jnp.float32)
            w2bd_ref[e * _H:(e + 1) * _H, :] = jax.lax.dot_general(
                w2t_ref[:, e, :], scat, (((0,), (0,)), ((), ())),
                preferred_element_type=jnp.float32).astype(jnp.bfloat16)
        b2t = b2_ref[...].T
        for c in range(_C):
            b2c_ref[:, c * _E:(c + 1) * _E] = b2t[c:c + 1, :]
        exp_of_lane = jax.lax.broadcasted_iota(jnp.int32, (_EC, _E), 0) % _E
        ecol = jax.lax.broadcasted_iota(jnp.int32, (_EC, _E), 1)
        bt_ref[...] = (exp_of_lane == ecol).astype(jnp.float32)
        erow = jax.lax.broadcasted_iota(jnp.int32, (_E, _EC), 0)
        lane_e = jax.lax.broadcasted_iota(jnp.int32, (_E, _EC), 1) % _E
        b_ref[...] = (erow == lane_e).astype(jnp.float32)
        cls_of_lane = jax.lax.broadcasted_iota(jnp.int32, (_EC, _C), 0) // _E
        ccol = jax.lax.broadcasted_iota(jnp.int32, (_EC, _C), 1)
        g_ref[...] = (cls_of_lane == ccol).astype(jnp.float32)

    for half in range(2):
        sl = pl.ds(half * _HB, _HB)
        x = x_ref[sl, :]

        glt = jax.lax.dot_general(rwt_ref[...], x, (((1,), (1,)), ((), ())),
                                  preferred_element_type=jnp.float32)
        glt = glt + rb_ref[...].T
        glt = glt - jnp.max(glt, axis=0, keepdims=True)
        get = jnp.exp(glt)
        gpt = get / jnp.sum(get, axis=0, keepdims=True)
        gpt_out_ref[:, sl] = gpt

        erow = jax.lax.broadcasted_iota(jnp.int32, (_E, _HB), 0)
        rankt = jnp.zeros((_E, _HB), dtype=jnp.int32)
        for ep in range(_E):
            row = gpt[ep:ep + 1, :]
            beats = (row > gpt) | ((row == gpt) & (ep < erow))
            rankt = rankt + beats.astype(jnp.int32)
        wselt = jnp.where(rankt < _TOPK, gpt, 0.0)
        wsel = (wselt / jnp.sum(wselt, axis=0, keepdims=True)).T

        mu = jnp.mean(x, axis=-1, keepdims=True)
        xc = x - mu
        var = jnp.mean(xc * xc, axis=-1, keepdims=True)
        xn = xc * jax.lax.rsqrt(var + _EPS)

        h = jnp.dot(xn.astype(jnp.bfloat16), w1s_ref[...],
                    preferred_element_type=jnp.float32)
        h = h + b1e_ref[...]
        h = 0.5 * h * (1.0 + jax.lax.erf(h * (1.0 / math.sqrt(2.0))))
        lo = jnp.dot(h.astype(jnp.bfloat16), w2bd_ref[...],
                     preferred_element_type=jnp.float32)
        lo = lo + b2c_ref[...]

        m = jnp.max(lo, axis=-1, keepdims=True)
        p = jnp.exp(lo - m)
        ssum = jnp.dot(p, bt_ref[...], preferred_element_type=jnp.float32)
        sr = 1.0 / ssum
        probs = p * jnp.dot(sr, b_ref[...], preferred_element_type=jnp.float32)
        pt = probs.T
        for c in range(_C):
            apt_ref[c, :, sl] = pt[c * _E:(c + 1) * _E, :]

        cwp = jnp.dot(wsel * sr, b_ref[...],
                      preferred_element_type=jnp.float32) * p
        wt_ref[:, sl] = jax.lax.dot_general(
            g_ref[...], cwp, (((0,), (1,)), ((), ())),
            preferred_element_type=jnp.float32)


@jax.jit
def kernel(x, router_w, router_b, ln_scale, ln_bias, w1, b1, w2, b2):
    rb2 = router_b.reshape(1, _E)
    grid = (_N // _BN,)
    out_shapes = (
        jax.ShapeDtypeStruct((_C, _N), jnp.float32),
        jax.ShapeDtypeStruct((_C, _E, _N), jnp.float32),
        jax.ShapeDtypeStruct((_E, _N), jnp.float32),
    )
    in_specs = [
        pl.BlockSpec((_BN, _D), lambda i: (i, 0)),
        pl.BlockSpec((_E, _D), lambda i: (0, 0)),
        pl.BlockSpec((1, _E), lambda i: (0, 0)),
        pl.BlockSpec((_E, _D), lambda i: (0, 0)),
        pl.BlockSpec((_E, _D), lambda i: (0, 0)),
        pl.BlockSpec((_E, _D, _H), lambda i: (0, 0, 0)),
        pl.BlockSpec((_E, _H), lambda i: (0, 0)),
        pl.BlockSpec((_C, _E, _H), lambda i: (0, 0, 0)),
        pl.BlockSpec((_E, _C), lambda i: (0, 0)),
    ]
    out_specs = (
        pl.BlockSpec((_C, _BN), lambda i: (0, i)),
        pl.BlockSpec((_C, _E, _BN), lambda i: (0, 0, i)),
        pl.BlockSpec((_E, _BN), lambda i: (0, i)),
    )
    scratch_shapes = [
        pltpu.VMEM((_D, _EH), jnp.bfloat16),
        pltpu.VMEM((1, _EH), jnp.float32),
        pltpu.VMEM((_EH, _EC), jnp.bfloat16),
        pltpu.VMEM((1, _EC), jnp.float32),
        pltpu.VMEM((_EC, _E), jnp.float32),
        pltpu.VMEM((_E, _EC), jnp.float32),
        pltpu.VMEM((_EC, _C), jnp.float32),
    ]
    wt, apt, gpt = pl.pallas_call(
        _moe_block_kernel,
        grid=grid,
        in_specs=in_specs,
        out_specs=out_specs,
        out_shape=out_shapes,
        scratch_shapes=scratch_shapes,
    )(x, router_w.T, rb2, ln_scale, ln_bias, w1, b1, w2.transpose(2, 0, 1), b2)
    return wt.T, apt.transpose(1, 2, 0), gpt.T


# speedup vs baseline: 2.0045x; 1.0234x over previous
import math

import jax
import jax.numpy as jnp
from jax.experimental import pallas as pl
from jax.experimental.pallas import tpu as pltpu

_N, _D, _E, _H, _C, _TOPK = 2048, 768, 16, 128, 10, 4
_EH = _E * _H
_EC = _E * _C
_EPS = 1e-5
_BN = 1024
_NH = 2
_HB = _BN // _NH


def _moe_block_kernel(x_ref, rwt_ref, rb_ref, lns_ref, lnb_ref, w1_ref, b1_ref,
                      w2t_ref, b2_ref, wt_ref, apt_ref, gpt_out_ref,
                      w1s_ref, b1e_ref, w2bd_ref, b2c_ref, bt_ref, b_ref, g_ref):
    @pl.when(pl.program_id(0) == 0)
    def _fold():
        lnst = lns_ref[...].T
        lane = jax.lax.broadcasted_iota(jnp.int32, (_C, _EC), 1)
        crow = jax.lax.broadcasted_iota(jnp.int32, (_C, _EC), 0)
        for e in range(_E):
            s = lnst[:, e:e + 1]
            w1s_ref[:, e * _H:(e + 1) * _H] = (s * w1_ref[e]).astype(jnp.bfloat16)
            tb = jnp.dot(lnb_ref[e].reshape(1, _D), w1_ref[e],
                         preferred_element_type=jnp.float32)
            b1e_ref[:, e * _H:(e + 1) * _H] = tb + b1_ref[e][None, :]
            scat = (lane == crow * _E + e).astype(jnp.float32)
            w2bd_ref[e * _H:(e + 1) * _H, :] = jax.lax.dot_general(
                w2t_ref[:, e, :], scat, (((0,), (0,)), ((), ())),
                preferred_element_type=jnp.float32).astype(jnp.bfloat16)
        b2t = b2_ref[...].T
        for c in range(_C):
            b2c_ref[:, c * _E:(c + 1) * _E] = b2t[c:c + 1, :]
        exp_of_lane = jax.lax.broadcasted_iota(jnp.int32, (_EC, _E), 0) % _E
        ecol = jax.lax.broadcasted_iota(jnp.int32, (_EC, _E), 1)
        bt_ref[...] = (exp_of_lane == ecol).astype(jnp.float32)
        erow = jax.lax.broadcasted_iota(jnp.int32, (_E, _EC), 0)
        lane_e = jax.lax.broadcasted_iota(jnp.int32, (_E, _EC), 1) % _E
        b_ref[...] = (erow == lane_e).astype(jnp.float32)
        cls_of_lane = jax.lax.broadcasted_iota(jnp.int32, (_EC, _C), 0) // _E
        ccol = jax.lax.broadcasted_iota(jnp.int32, (_EC, _C), 1)
        g_ref[...] = (cls_of_lane == ccol).astype(jnp.float32)

    for half in range(_NH):
        sl = pl.ds(half * _HB, _HB)
        x = x_ref[sl, :]

        glt = jax.lax.dot_general(rwt_ref[...], x, (((1,), (1,)), ((), ())),
                                  preferred_element_type=jnp.float32)
        glt = glt + rb_ref[...].T
        glt = glt - jnp.max(glt, axis=0, keepdims=True)
        get = jnp.exp(glt)
        gpt = get / jnp.sum(get, axis=0, keepdims=True)
        gpt_out_ref[:, sl] = gpt

        erow = jax.lax.broadcasted_iota(jnp.int32, (_E, _HB), 0)
        rankt = jnp.zeros((_E, _HB), dtype=jnp.int32)
        for ep in range(_E):
            row = gpt[ep:ep + 1, :]
            beats = (row > gpt) | ((row == gpt) & (ep < erow))
            rankt = rankt + beats.astype(jnp.int32)
        wselt = jnp.where(rankt < _TOPK, gpt, 0.0)
        wsel = (wselt / jnp.sum(wselt, axis=0, keepdims=True)).T

        mu = jnp.mean(x, axis=-1, keepdims=True)
        xc = x - mu
        var = jnp.mean(xc * xc, axis=-1, keepdims=True)
        xn = xc * jax.lax.rsqrt(var + _EPS)

        h = jnp.dot(xn.astype(jnp.bfloat16), w1s_ref[...],
                    preferred_element_type=jnp.float32)
        h = h + b1e_ref[...]
        h = 0.5 * h * (1.0 + jax.lax.erf(h * (1.0 / math.sqrt(2.0))))
        lo = jnp.dot(h.astype(jnp.bfloat16), w2bd_ref[...],
                     preferred_element_type=jnp.float32)
        lo = lo + b2c_ref[...]

        m = jnp.max(lo, axis=-1, keepdims=True)
        p = jnp.exp(lo - m)
        ssum = jnp.dot(p, bt_ref[...], preferred_element_type=jnp.float32)
        sr = 1.0 / ssum
        probs = p * jnp.dot(sr, b_ref[...], preferred_element_type=jnp.float32)
        pt = probs.T
        for c in range(_C):
            apt_ref[c, :, sl] = pt[c * _E:(c + 1) * _E, :]

        cwp = jnp.dot(wsel * sr, b_ref[...],
                      preferred_element_type=jnp.float32) * p
        wt_ref[:, sl] = jax.lax.dot_general(
            g_ref[...], cwp, (((0,), (1,)), ((), ())),
            preferred_element_type=jnp.float32)


@jax.jit
def kernel(x, router_w, router_b, ln_scale, ln_bias, w1, b1, w2, b2):
    rb2 = router_b.reshape(1, _E)
    grid = (_N // _BN,)
    out_shapes = (
        jax.ShapeDtypeStruct((_C, _N), jnp.float32),
        jax.ShapeDtypeStruct((_C, _E, _N), jnp.float32),
        jax.ShapeDtypeStruct((_E, _N), jnp.float32),
    )
    in_specs = [
        pl.BlockSpec((_BN, _D), lambda i: (i, 0)),
        pl.BlockSpec((_E, _D), lambda i: (0, 0)),
        pl.BlockSpec((1, _E), lambda i: (0, 0)),
        pl.BlockSpec((_E, _D), lambda i: (0, 0)),
        pl.BlockSpec((_E, _D), lambda i: (0, 0)),
        pl.BlockSpec((_E, _D, _H), lambda i: (0, 0, 0)),
        pl.BlockSpec((_E, _H), lambda i: (0, 0)),
        pl.BlockSpec((_C, _E, _H), lambda i: (0, 0, 0)),
        pl.BlockSpec((_E, _C), lambda i: (0, 0)),
    ]
    out_specs = (
        pl.BlockSpec((_C, _BN), lambda i: (0, i)),
        pl.BlockSpec((_C, _E, _BN), lambda i: (0, 0, i)),
        pl.BlockSpec((_E, _BN), lambda i: (0, i)),
    )
    scratch_shapes = [
        pltpu.VMEM((_D, _EH), jnp.bfloat16),
        pltpu.VMEM((1, _EH), jnp.float32),
        pltpu.VMEM((_EH, _EC), jnp.bfloat16),
        pltpu.VMEM((1, _EC), jnp.float32),
        pltpu.VMEM((_EC, _E), jnp.float32),
        pltpu.VMEM((_E, _EC), jnp.float32),
        pltpu.VMEM((_EC, _C), jnp.float32),
    ]
    wt, apt, gpt = pl.pallas_call(
        _moe_block_kernel,
        grid=grid,
        in_specs=in_specs,
        out_specs=out_specs,
        out_shape=out_shapes,
        scratch_shapes=scratch_shapes,
    )(x, router_w.T, rb2, ln_scale, ln_bias, w1, b1, w2.transpose(2, 0, 1), b2)
    return wt.T, apt.transpose(1, 2, 0), gpt.T
